# Initial kernel scaffold; baseline (speedup 1.0000x reference)
import jax
import jax.numpy as jnp
from jax.experimental import pallas as pl


def kernel(x, edge_index, W1, b1, W2, b2):
    raise NotImplementedError("write your pallas kernel here")


# speedup vs baseline: 8.6469x; 8.6469x over previous
import functools

import jax
import jax.numpy as jnp
from jax import lax
from jax.experimental import pallas as pl
from jax.experimental.pallas import tpu as pltpu
from jax.experimental.pallas import tpu_sc as plsc

_N = 10000
_D = 128
_N_P = 10240
_E = 320000
_E_P = 327680
_LANES = 128
_EROWS = _E_P // _LANES
_NC = 2
_NS = 16
_TILES = _NC * _NS
_EROWS_PT = _EROWS // _TILES
_EROWS_PS = _EROWS // _NS
_NROWS = _N_P // _LANES
_ACC_PT = _N_P // _NS
_TRASH = _N

_mesh = plsc.VectorSubcoreMesh(core_axis_name="c", subcore_axis_name="s")


@functools.partial(
    pl.kernel,
    out_type=jax.ShapeDtypeStruct((_TILES * _N_P,), jnp.float32),
    mesh=_mesh,
    compiler_params=pltpu.CompilerParams(needs_layout_passes=False),
    scratch_types=[
        pltpu.VMEM((_EROWS_PT, _LANES), jnp.int32),
        pltpu.VMEM((_N_P,), jnp.float32),
        pltpu.SemaphoreType.DMA,
    ],
)
def _deg_kernel(dst_hbm, out_hbm, dst_v, cnt_v, sem):
    cid = lax.axis_index("c")
    sid = lax.axis_index("s")
    tid = cid * _NS + sid
    zero16 = jnp.zeros((16,), jnp.float32)

    def _zcnt(i, _):
        cnt_v[pl.ds(i * 16, 16)] = zero16
        return 0
    lax.fori_loop(0, _N_P // 16, _zcnt, 0)

    pltpu.sync_copy(dst_hbm.at[pl.ds(tid * _EROWS_PT, _EROWS_PT)], dst_v)

    ones16 = jnp.ones((16,), jnp.float32)

    def _cnt(i, _):
        r = i // 8
        k = i - r * 8
        idx = dst_v[r, pl.ds(k * 16, 16)]
        plsc.addupdate_scatter(cnt_v, [idx], ones16)
        return 0
    lax.fori_loop(0, _EROWS_PT * 8, _cnt, 0)

    pltpu.sync_copy(cnt_v, out_hbm.at[pl.ds(tid * _N_P, _N_P)])


def _tc_deg_reduce(degp):
    def body(p_ref, o_ref):
        o_ref[...] = jnp.sum(p_ref[...], axis=0)

    return pl.pallas_call(
        body,
        out_shape=jax.ShapeDtypeStruct((_NROWS, _LANES), jnp.float32),
    )(degp)


@functools.partial(
    pl.kernel,
    out_type=jax.ShapeDtypeStruct((_NC, _N_P, _D), jnp.float32),
    mesh=_mesh,
    compiler_params=pltpu.CompilerParams(needs_layout_passes=False),
    scratch_types=[
        pltpu.VMEM((_EROWS_PT, _LANES), jnp.int32),
        pltpu.VMEM((_EROWS_PT, _LANES), jnp.int32),
        pltpu.VMEM((_LANES, _D), jnp.float32),
        pltpu.VMEM((32, _D), jnp.float32),
        pltpu.VMEM_SHARED((_N_P, _D), jnp.float32),
        pltpu.SemaphoreType.DMA,
    ],
)
def _agg_kernel(g_hbm, src_hbm, dst_hbm, out_hbm,
                src_v, dst_v, buf, zb_v, acc_sh, sem):
    cid = lax.axis_index("c")
    sid = lax.axis_index("s")
    tid = cid * _NS + sid
    zero16 = jnp.zeros((16,), jnp.float32)

    def _z(i, _):
        r = i // 8
        k = i - r * 8
        zb_v[r, pl.ds(k * 16, 16)] = zero16
        return 0
    lax.fori_loop(0, 32 * 8, _z, 0)

    def _zs(b, _):
        pltpu.sync_copy(zb_v, acc_sh.at[pl.ds(sid * _ACC_PT + b * 32, 32)])
        return 0
    lax.fori_loop(0, _ACC_PT // 32, _zs, 0)

    pltpu.sync_copy(src_hbm.at[pl.ds(tid * _EROWS_PT, _EROWS_PT)], src_v)
    pltpu.sync_copy(dst_hbm.at[pl.ds(tid * _EROWS_PT, _EROWS_PT)], dst_v)
    plsc.subcore_barrier()

    def _body(i, _):
        pltpu.async_copy(g_hbm.at[src_v.at[i]], buf, sem).wait()
        pltpu.sync_copy(buf, acc_sh.at[dst_v.at[i]], add=True)
        return 0
    lax.fori_loop(0, _EROWS_PT, _body, 0)

    plsc.subcore_barrier()
    pltpu.sync_copy(acc_sh.at[pl.ds(sid * _ACC_PT, _ACC_PT)],
                    out_hbm.at[cid, pl.ds(sid * _ACC_PT, _ACC_PT)])


_BM = 1024
_BM3 = 1000


def _tc_layer1(x_pad, w1, deg):
    def body(x_ref, w_ref, d_ref, g_ref):
        dis = lax.rsqrt(d_ref[...] + 1.0)
        h = jnp.dot(x_ref[...], w_ref[...], preferred_element_type=jnp.float32)
        g_ref[...] = h * dis

    return pl.pallas_call(
        body,
        grid=(_N_P // _BM,),
        in_specs=[
            pl.BlockSpec((_BM, _D), lambda b: (b, 0)),
            pl.BlockSpec((_D, _D), lambda b: (0, 0)),
            pl.BlockSpec((_BM, 1), lambda b: (b, 0)),
        ],
        out_specs=pl.BlockSpec((_BM, _D), lambda b: (b, 0)),
        out_shape=jax.ShapeDtypeStruct((_N_P, _D), jnp.float32),
    )(x_pad, w1, deg)


def _tc_layer2(parts, g1, deg, w2, b1):
    def body(p_ref, g_ref, d_ref, w_ref, b_ref, o_ref):
        dis = lax.rsqrt(d_ref[...] + 1.0)
        s = p_ref[0] + p_ref[1] + g_ref[...]
        z = jnp.maximum(s * dis + b_ref[...], 0.0)
        o_ref[...] = jnp.dot(z, w_ref[...], preferred_element_type=jnp.float32) * dis

    return pl.pallas_call(
        body,
        grid=(_N_P // _BM,),
        in_specs=[
            pl.BlockSpec((_NC, _BM, _D), lambda b: (0, b, 0)),
            pl.BlockSpec((_BM, _D), lambda b: (b, 0)),
            pl.BlockSpec((_BM, 1), lambda b: (b, 0)),
            pl.BlockSpec((_D, _D), lambda b: (0, 0)),
            pl.BlockSpec((1, _D), lambda b: (0, 0)),
        ],
        out_specs=pl.BlockSpec((_BM, _D), lambda b: (b, 0)),
        out_shape=jax.ShapeDtypeStruct((_N_P, _D), jnp.float32),
    )(parts, g1, deg, w2, b1)


def _tc_layer3(parts, g2, deg, b2):
    def body(p_ref, g_ref, d_ref, b_ref, o_ref):
        dis = lax.rsqrt(d_ref[...] + 1.0)
        s = p_ref[0] + p_ref[1] + g_ref[...]
        o_ref[...] = s * dis + b_ref[...]

    return pl.pallas_call(
        body,
        grid=(_N // _BM3,),
        in_specs=[
            pl.BlockSpec((_NC, _BM3, _D), lambda b: (0, b, 0)),
            pl.BlockSpec((_BM3, _D), lambda b: (b, 0)),
            pl.BlockSpec((_BM3, 1), lambda b: (b, 0)),
            pl.BlockSpec((1, _D), lambda b: (0, 0)),
        ],
        out_specs=pl.BlockSpec((_BM3, _D), lambda b: (b, 0)),
        out_shape=jax.ShapeDtypeStruct((_N, _D), jnp.float32),
    )(parts, g2, deg, b2)


def kernel(x, edge_index, W1, b1, W2, b2):
    x = x.astype(jnp.float32)
    src = edge_index[0].astype(jnp.int32)
    dst = edge_index[1].astype(jnp.int32)
    src_p = jnp.concatenate(
        [src, jnp.zeros((_E_P - _E,), jnp.int32)]).reshape(_EROWS, _LANES)
    dst_p = jnp.concatenate(
        [dst, jnp.full((_E_P - _E,), _TRASH, jnp.int32)]).reshape(_EROWS, _LANES)
    x_pad = jnp.pad(x, ((0, _N_P - _N), (0, 0)))

    degp = _deg_kernel(dst_p).reshape(_TILES, _NROWS, _LANES)
    deg = _tc_deg_reduce(degp).reshape(_N_P, 1)
    g1 = _tc_layer1(x_pad, W1, deg)
    p = _agg_kernel(g1, src_p, dst_p)
    g2 = _tc_layer2(p, g1, deg, W2, b1.reshape(1, _D))
    q = _agg_kernel(g2, src_p, dst_p)
    return _tc_layer3(q, g2, deg, b2.reshape(1, _D))


# speedup vs baseline: 9.6723x; 1.1186x over previous
import functools

import jax
import jax.numpy as jnp
from jax import lax
from jax.experimental import pallas as pl
from jax.experimental.pallas import tpu as pltpu
from jax.experimental.pallas import tpu_sc as plsc

_N = 10000
_D = 128
_N_P = 10240
_E = 320000
_E_P = 327680
_LANES = 128
_EROWS = _E_P // _LANES
_NC = 2
_NS = 16
_TILES = _NC * _NS
_EROWS_PT = _EROWS // _TILES
_EROWS_PS = _EROWS // _NS
_NROWS = _N_P // _LANES
_ACC_PT = _N_P // _NS
_TRASH = _N

_mesh = plsc.VectorSubcoreMesh(core_axis_name="c", subcore_axis_name="s")


@functools.partial(
    pl.kernel,
    out_type=jax.ShapeDtypeStruct((_TILES * _N_P,), jnp.float32),
    mesh=_mesh,
    compiler_params=pltpu.CompilerParams(needs_layout_passes=False),
    scratch_types=[
        pltpu.VMEM((_EROWS_PT, _LANES), jnp.int32),
        pltpu.VMEM((_N_P,), jnp.float32),
        pltpu.SemaphoreType.DMA,
    ],
)
def _deg_kernel(dst_hbm, out_hbm, dst_v, cnt_v, sem):
    cid = lax.axis_index("c")
    sid = lax.axis_index("s")
    tid = cid * _NS + sid
    zero16 = jnp.zeros((16,), jnp.float32)

    def _zcnt(i, _):
        cnt_v[pl.ds(i * 16, 16)] = zero16
        return 0
    lax.fori_loop(0, _N_P // 16, _zcnt, 0)

    pltpu.sync_copy(dst_hbm.at[pl.ds(tid * _EROWS_PT, _EROWS_PT)], dst_v)

    ones16 = jnp.ones((16,), jnp.float32)

    def _cnt(i, _):
        r = i // 8
        k = i - r * 8
        idx = dst_v[r, pl.ds(k * 16, 16)]
        plsc.addupdate_scatter(cnt_v, [idx], ones16)
        return 0
    lax.fori_loop(0, _EROWS_PT * 8, _cnt, 0)

    pltpu.sync_copy(cnt_v, out_hbm.at[pl.ds(tid * _N_P, _N_P)])


def _tc_deg_reduce(degp):
    def body(p_ref, o_ref):
        o_ref[...] = jnp.sum(p_ref[...], axis=0)

    return pl.pallas_call(
        body,
        out_shape=jax.ShapeDtypeStruct((_NROWS, _LANES), jnp.float32),
    )(degp)


@functools.partial(
    pl.kernel,
    out_type=jax.ShapeDtypeStruct((_NC, _N_P, _D), jnp.float32),
    mesh=_mesh,
    compiler_params=pltpu.CompilerParams(needs_layout_passes=False),
    scratch_types=[
        pltpu.VMEM((_EROWS_PT // 2, _LANES), jnp.int32),
        pltpu.VMEM((_EROWS_PT // 2, _LANES), jnp.int32),
        pltpu.VMEM((_LANES, _D), jnp.float32),
        pltpu.VMEM((_LANES, _D), jnp.float32),
        pltpu.VMEM_SHARED((_N_P, _D), jnp.float32),
        pltpu.SemaphoreType.DMA,
        pltpu.SemaphoreType.DMA,
        pltpu.SemaphoreType.DMA,
        pltpu.SemaphoreType.DMA,
    ],
)
def _agg_kernel(g_hbm, src_hbm, dst_hbm, out_hbm,
                src_v, dst_v, bufa, bufb, acc_sh, gsa, gsb, ssa, ssb):
    cid = lax.axis_index("c")
    sid = lax.axis_index("s")
    tid = cid * _NS + sid
    zero16 = jnp.zeros((16,), jnp.float32)

    def _z(i, _):
        r = i // 8
        k = i - r * 8
        bufa[r, pl.ds(k * 16, 16)] = zero16
        return 0
    lax.fori_loop(0, _LANES * 8, _z, 0)

    def _zs(b, _):
        pltpu.sync_copy(bufa, acc_sh.at[pl.ds(sid * _ACC_PT + b * _LANES, _LANES)])
        return 0
    lax.fori_loop(0, _ACC_PT // _LANES, _zs, 0)
    plsc.subcore_barrier()

    nh = _EROWS_PT // 2
    for h in range(2):
        base = tid * _EROWS_PT + h * nh
        pltpu.sync_copy(src_hbm.at[pl.ds(base, nh)], src_v)
        pltpu.sync_copy(dst_hbm.at[pl.ds(base, nh)], dst_v)
        pltpu.async_copy(g_hbm.at[src_v.at[0]], bufa, gsa)

        def _pair(j, _):
            pltpu.async_copy(g_hbm.at[src_v.at[2 * j + 1]], bufb, gsb)
            pltpu.make_async_copy(g_hbm.at[src_v.at[0]], bufa, gsa).wait()
            pltpu.async_copy(bufa, acc_sh.at[dst_v.at[2 * j]], ssa, add=True)
            pltpu.make_async_copy(bufa, acc_sh.at[dst_v.at[2 * j]], ssa).wait()

            @pl.when(j < nh // 2 - 1)
            def _():
                pltpu.async_copy(g_hbm.at[src_v.at[2 * j + 2]], bufa, gsa)

            pltpu.make_async_copy(g_hbm.at[src_v.at[0]], bufb, gsb).wait()
            pltpu.async_copy(bufb, acc_sh.at[dst_v.at[2 * j + 1]], ssb, add=True)
            pltpu.make_async_copy(bufb, acc_sh.at[dst_v.at[2 * j + 1]], ssb).wait()
            return 0
        lax.fori_loop(0, nh // 2, _pair, 0)

    plsc.subcore_barrier()
    pltpu.sync_copy(acc_sh.at[pl.ds(sid * _ACC_PT, _ACC_PT)],
                    out_hbm.at[cid, pl.ds(sid * _ACC_PT, _ACC_PT)])


_BM = 1024
_BM3 = 1000


def _tc_layer1(x_pad, w1, deg):
    def body(x_ref, w_ref, d_ref, g_ref):
        dis = lax.rsqrt(d_ref[...] + 1.0)
        h = jnp.dot(x_ref[...], w_ref[...], preferred_element_type=jnp.float32)
        g_ref[...] = h * dis

    return pl.pallas_call(
        body,
        grid=(_N_P // _BM,),
        in_specs=[
            pl.BlockSpec((_BM, _D), lambda b: (b, 0)),
            pl.BlockSpec((_D, _D), lambda b: (0, 0)),
            pl.BlockSpec((_BM, 1), lambda b: (b, 0)),
        ],
        out_specs=pl.BlockSpec((_BM, _D), lambda b: (b, 0)),
        out_shape=jax.ShapeDtypeStruct((_N_P, _D), jnp.float32),
    )(x_pad, w1, deg)


def _tc_layer2(parts, g1, deg, w2, b1):
    def body(p_ref, g_ref, d_ref, w_ref, b_ref, o_ref):
        dis = lax.rsqrt(d_ref[...] + 1.0)
        s = p_ref[0] + p_ref[1] + g_ref[...]
        z = jnp.maximum(s * dis + b_ref[...], 0.0)
        o_ref[...] = jnp.dot(z, w_ref[...], preferred_element_type=jnp.float32) * dis

    return pl.pallas_call(
        body,
        grid=(_N_P // _BM,),
        in_specs=[
            pl.BlockSpec((_NC, _BM, _D), lambda b: (0, b, 0)),
            pl.BlockSpec((_BM, _D), lambda b: (b, 0)),
            pl.BlockSpec((_BM, 1), lambda b: (b, 0)),
            pl.BlockSpec((_D, _D), lambda b: (0, 0)),
            pl.BlockSpec((1, _D), lambda b: (0, 0)),
        ],
        out_specs=pl.BlockSpec((_BM, _D), lambda b: (b, 0)),
        out_shape=jax.ShapeDtypeStruct((_N_P, _D), jnp.float32),
    )(parts, g1, deg, w2, b1)


def _tc_layer3(parts, g2, deg, b2):
    def body(p_ref, g_ref, d_ref, b_ref, o_ref):
        dis = lax.rsqrt(d_ref[...] + 1.0)
        s = p_ref[0] + p_ref[1] + g_ref[...]
        o_ref[...] = s * dis + b_ref[...]

    return pl.pallas_call(
        body,
        grid=(_N // _BM3,),
        in_specs=[
            pl.BlockSpec((_NC, _BM3, _D), lambda b: (0, b, 0)),
            pl.BlockSpec((_BM3, _D), lambda b: (b, 0)),
            pl.BlockSpec((_BM3, 1), lambda b: (b, 0)),
            pl.BlockSpec((1, _D), lambda b: (0, 0)),
        ],
        out_specs=pl.BlockSpec((_BM3, _D), lambda b: (b, 0)),
        out_shape=jax.ShapeDtypeStruct((_N, _D), jnp.float32),
    )(parts, g2, deg, b2)


def kernel(x, edge_index, W1, b1, W2, b2):
    x = x.astype(jnp.float32)
    src = edge_index[0].astype(jnp.int32)
    dst = edge_index[1].astype(jnp.int32)
    src_p = jnp.concatenate(
        [src, jnp.zeros((_E_P - _E,), jnp.int32)]).reshape(_EROWS, _LANES)
    dst_p = jnp.concatenate(
        [dst, jnp.full((_E_P - _E,), _TRASH, jnp.int32)]).reshape(_EROWS, _LANES)
    x_pad = jnp.pad(x, ((0, _N_P - _N), (0, 0)))

    degp = _deg_kernel(dst_p).reshape(_TILES, _NROWS, _LANES)
    deg = _tc_deg_reduce(degp).reshape(_N_P, 1)
    g1 = _tc_layer1(x_pad, W1, deg)
    p = _agg_kernel(g1, src_p, dst_p)
    g2 = _tc_layer2(p, g1, deg, W2, b1.reshape(1, _D))
    q = _agg_kernel(g2, src_p, dst_p)
    return _tc_layer3(q, g2, deg, b2.reshape(1, _D))
